# baseline (speedup 1.0000x reference)
import functools

import jax
import jax.numpy as jnp
from jax import lax
from jax.experimental import pallas as pl
from jax.experimental.pallas import tpu as pltpu
from jax.experimental.pallas import tpu_sc as plsc

T = 2048
H = 1024
II = 1024
E = 8
K = 2
TK = T * K
TM = 128
NT = (TK + E * TM) // TM
RP = NT * TM
ALPHA = 1.702
BETA = 1.0

NC = 2
NS = 16
NW = NC * NS
GPT = RP // NW
TPT = T // NW


def _routing_body(rl_ref, pos_ref, wts_ref, te_ref):
    lt = rl_ref[...]
    ei = lax.broadcasted_iota(jnp.int32, (E, T), 0)
    m1 = jnp.max(lt, axis=0, keepdims=True)
    i1 = jnp.min(jnp.where(lt == m1, ei, E), axis=0, keepdims=True)
    l2 = jnp.where(ei == i1, -jnp.inf, lt)
    m2 = jnp.max(l2, axis=0, keepdims=True)
    i2 = jnp.min(jnp.where(l2 == m2, ei, E), axis=0, keepdims=True)
    w1 = jax.nn.sigmoid(m1 - m2)

    sel1 = (ei == i1).astype(jnp.float32)
    sel2 = (ei == i2).astype(jnp.float32)
    oht = sel1 + sel2
    utri = (lax.broadcasted_iota(jnp.int32, (T, T), 0)
            < lax.broadcasted_iota(jnp.int32, (T, T), 1)).astype(jnp.float32)
    pref = lax.dot_general(oht, utri, (((1,), (0,)), ((), ())),
                           preferred_element_type=jnp.float32)
    counts = jnp.sum(oht, axis=1, keepdims=True)
    pc = jnp.ceil(counts / TM) * TM
    ltri8 = (lax.broadcasted_iota(jnp.int32, (E, E), 0)
             > lax.broadcasted_iota(jnp.int32, (E, E), 1)).astype(jnp.float32)
    gs = lax.dot_general(ltri8, pc, (((1,), (0,)), ((), ())),
                         preferred_element_type=jnp.float32)
    ge = gs + pc

    base = gs + pref
    pos0 = jnp.sum(sel1 * base, axis=0, keepdims=True)
    pos1 = jnp.sum(sel2 * base, axis=0, keepdims=True)
    pos_ref[...] = jnp.concatenate([pos0, pos1], axis=0).astype(jnp.int32)
    wts_ref[...] = jnp.concatenate([w1, 1.0 - w1], axis=0)

    jt = (lax.broadcasted_iota(jnp.int32, (E, NT), 1) * TM).astype(jnp.float32)
    te = jnp.sum((jt >= ge).astype(jnp.float32), axis=0, keepdims=True)
    te_ref[...] = jnp.minimum(te, E - 1).astype(jnp.int32)


def _routing(router_logits_t):
    return pl.pallas_call(
        _routing_body,
        out_shape=(
            jax.ShapeDtypeStruct((K, T), jnp.int32),
            jax.ShapeDtypeStruct((K, T), jnp.float32),
            jax.ShapeDtypeStruct((1, NT), jnp.int32),
        ),
    )(router_logits_t)


def _dispatch_body(pos_hbm, wts_hbm, tok_hbm, scale_hbm,
                   pos_v, wts_v, tok_v, scl_v, semi, semw):
    wid = lax.axis_index("c") * NS + lax.axis_index("s")
    lo = wid * GPT
    cpp = pltpu.async_copy(pos_hbm, pos_v, semi)
    cpw = pltpu.async_copy(wts_hbm, wts_v, semi)

    zi = jnp.zeros((16,), jnp.int32)
    zf = jnp.zeros((16,), jnp.float32)
    for q in range(GPT // 16):
        tok_v[pl.ds(q * 16, 16)] = zi
        scl_v[pl.ds(q * 16, 16)] = zf
    cpp.wait()
    cpw.wait()

    lane = lax.broadcasted_iota(jnp.int32, (16,), 0)

    @pl.loop(0, TK // 16, unroll=16)
    def _scatter(q):
        sr = pos_v[pl.ds(q * 16, 16)]
        tv = (q * 16 + lane) & (T - 1)
        m = (sr >= lo) & (sr < lo + GPT)
        li = sr - lo
        plsc.store_scatter(tok_v, [li], tv, mask=m)
        plsc.store_scatter(scl_v, [li], wts_v[pl.ds(q * 16, 16)], mask=m)

    w0 = pltpu.async_copy(tok_v, tok_hbm.at[pl.ds(lo, GPT)], semw)
    w1 = pltpu.async_copy(scl_v, scale_hbm.at[pl.ds(lo, GPT)], semw)
    w0.wait()
    w1.wait()


def _dispatch(pos_flat, wts_flat):
    mesh = plsc.VectorSubcoreMesh(core_axis_name="c", subcore_axis_name="s")
    f = pl.kernel(
        _dispatch_body,
        out_type=(
            jax.ShapeDtypeStruct((RP,), jnp.int32),
            jax.ShapeDtypeStruct((RP,), jnp.float32),
        ),
        mesh=mesh,
        scratch_types=(
            pltpu.VMEM((TK,), jnp.int32),
            pltpu.VMEM((TK,), jnp.float32),
            pltpu.VMEM((GPT,), jnp.int32),
            pltpu.VMEM((GPT,), jnp.float32),
            pltpu.SemaphoreType.DMA,
            pltpu.SemaphoreType.DMA,
        ),
        compiler_params=pltpu.CompilerParams(needs_layout_passes=False),
    )
    return f(pos_flat, wts_flat)


def _mm_body(te_ref, tok_ref, scale_ref, x_ref, w13_ref, w2_ref, b13_ref,
             b2_ref, out_ref, w13b_ref, w2b_ref):
    i = pl.program_id(0)
    prev = te_ref[jnp.maximum(i - 1, 0)]

    @pl.when((i == 0) | (te_ref[i] != prev))
    def _cast_weights():
        w13b_ref[...] = w13_ref[0].astype(jnp.bfloat16)
        w2b_ref[...] = w2_ref[0].astype(jnp.bfloat16)

    sel = (tok_ref[...] == lax.broadcasted_iota(jnp.int32, (TM, T), 1))
    xs = lax.dot_general(sel.astype(jnp.bfloat16), x_ref[...],
                         (((1,), (0,)), ((), ())),
                         preferred_element_type=jnp.float32)
    h = lax.dot_general(xs.astype(jnp.bfloat16), w13b_ref[...],
                        (((1,), (1,)), ((), ())),
                        preferred_element_type=jnp.float32)
    h = h + b13_ref[0]
    gate = h[:, :II]
    up = h[:, II:]
    act = gate * jax.nn.sigmoid(ALPHA * gate) * (up + BETA)
    y = lax.dot_general(act.astype(jnp.bfloat16), w2b_ref[...],
                        (((1,), (1,)), ((), ())),
                        preferred_element_type=jnp.float32)
    out_ref[...] = (y + b2_ref[0]) * scale_ref[...]


def _mm(te, tok, scale, x_bf, w13, w2, w13_bias, w2_bias):
    grid_spec = pltpu.PrefetchScalarGridSpec(
        num_scalar_prefetch=1,
        grid=(NT,),
        in_specs=[
            pl.BlockSpec((TM, 1), lambda i, te_r: (i, 0)),
            pl.BlockSpec((TM, 1), lambda i, te_r: (i, 0)),
            pl.BlockSpec((T, H), lambda i, te_r: (0, 0)),
            pl.BlockSpec((1, 2 * II, H), lambda i, te_r: (te_r[i], 0, 0)),
            pl.BlockSpec((1, H, II), lambda i, te_r: (te_r[i], 0, 0)),
            pl.BlockSpec((1, 1, 2 * II), lambda i, te_r: (te_r[i], 0, 0)),
            pl.BlockSpec((1, 1, H), lambda i, te_r: (te_r[i], 0, 0)),
        ],
        out_specs=pl.BlockSpec((TM, H), lambda i, te_r: (i, 0)),
        scratch_shapes=[
            pltpu.VMEM((2 * II, H), jnp.bfloat16),
            pltpu.VMEM((H, II), jnp.bfloat16),
        ],
    )
    return pl.pallas_call(
        _mm_body,
        grid_spec=grid_spec,
        out_shape=jax.ShapeDtypeStruct((RP, H), jnp.float32),
    )(te, tok, scale, x_bf, w13, w2, w13_bias.reshape(E, 1, 2 * II),
      w2_bias.reshape(E, 1, H))


def _combine_body(pos_hbm, ys_hbm, out_hbm, pv0, pv1, i0a, i0b, i1a, i1b,
                  b0a, b0b, b1a, b1b, semg, semw):
    wid = lax.axis_index("c") * NS + lax.axis_index("s")
    tok0 = wid * TPT
    pltpu.sync_copy(pos_hbm.at[pl.ds(tok0, TPT)], pv0)
    pltpu.sync_copy(pos_hbm.at[pl.ds(T + tok0, TPT)], pv1)
    NCH = TPT // 16
    b0 = [b0a, b0b]
    b1 = [b1a, b1b]
    i0 = [i0a, i0b]
    i1 = [i1a, i1b]
    g0 = [None] * NCH
    g1 = [None] * NCH
    w = [None] * NCH

    def issue(ch):
        p = ch % 2
        i0[p][...] = pv0[pl.ds(ch * 16, 16)]
        i1[p][...] = pv1[pl.ds(ch * 16, 16)]
        g0[ch] = pltpu.async_copy(ys_hbm.at[i0[p]], b0[p], semg)
        g1[ch] = pltpu.async_copy(ys_hbm.at[i1[p]], b1[p], semg)

    issue(0)
    issue(1)
    for ch in range(NCH):
        p = ch % 2
        if ch >= 2:
            w[ch - 2].wait()
            issue(ch)
        g0[ch].wait()
        g1[ch].wait()
        bb0 = b0[p]
        bb1 = b1[p]

        @pl.loop(0, 16 * (H // 16), unroll=8)
        def _add(q):
            r = lax.shift_right_logical(q, 6)
            c = lax.rem(q, H // 16)
            bb0[r, pl.ds(c * 16, 16)] = (bb0[r, pl.ds(c * 16, 16)]
                                         + bb1[r, pl.ds(c * 16, 16)])

        w[ch] = pltpu.async_copy(bb0, out_hbm.at[pl.ds(tok0 + ch * 16, 16)],
                                 semw)
    w[NCH - 2].wait()
    w[NCH - 1].wait()


def _combine(pos_flat, ys):
    mesh = plsc.VectorSubcoreMesh(core_axis_name="c", subcore_axis_name="s")
    f = pl.kernel(
        _combine_body,
        out_type=jax.ShapeDtypeStruct((T, H), jnp.float32),
        mesh=mesh,
        scratch_types=(
            pltpu.VMEM((TPT,), jnp.int32),
            pltpu.VMEM((TPT,), jnp.int32),
            pltpu.VMEM((16,), jnp.int32),
            pltpu.VMEM((16,), jnp.int32),
            pltpu.VMEM((16,), jnp.int32),
            pltpu.VMEM((16,), jnp.int32),
            pltpu.VMEM((16, H), jnp.float32),
            pltpu.VMEM((16, H), jnp.float32),
            pltpu.VMEM((16, H), jnp.float32),
            pltpu.VMEM((16, H), jnp.float32),
            pltpu.SemaphoreType.DMA,
            pltpu.SemaphoreType.DMA,
        ),
        compiler_params=pltpu.CompilerParams(needs_layout_passes=False),
    )
    return f(pos_flat, ys)


def kernel(x, router_logits, w13, w2, w13_bias, w2_bias):
    pos, wts, te = _routing(router_logits.T)
    pos_flat = pos.reshape(TK)
    wts_flat = wts.reshape(TK)
    tok, scale = _dispatch(pos_flat, wts_flat)
    ys = _mm(te.reshape(NT), tok.reshape(RP, 1), scale.reshape(RP, 1),
             x.astype(jnp.bfloat16), w13, w2, w13_bias, w2_bias)
    return _combine(pos_flat, ys)


# speedup vs baseline: 1.2826x; 1.2826x over previous
import functools

import jax
import jax.numpy as jnp
from jax import lax
from jax.experimental import pallas as pl
from jax.experimental.pallas import tpu as pltpu
from jax.experimental.pallas import tpu_sc as plsc

T = 2048
H = 1024
II = 1024
E = 8
K = 2
TK = T * K
TM = 256
NT = (TK + E * TM) // TM
RP = NT * TM
ALPHA = 1.702
BETA = 1.0

NC = 2
NS = 16
NW = NC * NS
GPT = RP // NW
TPT = T // NW


def _routing_body(rl_ref, pos_ref, wts_ref, te_ref):
    lt = rl_ref[...]
    ei = lax.broadcasted_iota(jnp.int32, (E, T), 0)
    m1 = jnp.max(lt, axis=0, keepdims=True)
    i1 = jnp.min(jnp.where(lt == m1, ei, E), axis=0, keepdims=True)
    l2 = jnp.where(ei == i1, -jnp.inf, lt)
    m2 = jnp.max(l2, axis=0, keepdims=True)
    i2 = jnp.min(jnp.where(l2 == m2, ei, E), axis=0, keepdims=True)
    w1 = jax.nn.sigmoid(m1 - m2)

    sel1 = (ei == i1).astype(jnp.float32)
    sel2 = (ei == i2).astype(jnp.float32)
    oht = sel1 + sel2
    utri = (lax.broadcasted_iota(jnp.int32, (T, T), 0)
            < lax.broadcasted_iota(jnp.int32, (T, T), 1)).astype(jnp.float32)
    pref = lax.dot_general(oht, utri, (((1,), (0,)), ((), ())),
                           preferred_element_type=jnp.float32)
    counts = jnp.sum(oht, axis=1, keepdims=True)
    pc = jnp.ceil(counts / TM) * TM
    ltri8 = (lax.broadcasted_iota(jnp.int32, (E, E), 0)
             > lax.broadcasted_iota(jnp.int32, (E, E), 1)).astype(jnp.float32)
    gs = lax.dot_general(ltri8, pc, (((1,), (0,)), ((), ())),
                         preferred_element_type=jnp.float32)
    ge = gs + pc

    base = gs + pref
    pos0 = jnp.sum(sel1 * base, axis=0, keepdims=True)
    pos1 = jnp.sum(sel2 * base, axis=0, keepdims=True)
    pos_ref[...] = jnp.concatenate([pos0, pos1], axis=0).astype(jnp.int32)
    wts_ref[...] = jnp.concatenate([w1, 1.0 - w1], axis=0)

    jt = (lax.broadcasted_iota(jnp.int32, (E, NT), 1) * TM).astype(jnp.float32)
    te = jnp.sum((jt >= ge).astype(jnp.float32), axis=0, keepdims=True)
    te_ref[...] = jnp.minimum(te, E - 1).astype(jnp.int32)


def _routing(router_logits_t):
    return pl.pallas_call(
        _routing_body,
        out_shape=(
            jax.ShapeDtypeStruct((K, T), jnp.int32),
            jax.ShapeDtypeStruct((K, T), jnp.float32),
            jax.ShapeDtypeStruct((1, NT), jnp.int32),
        ),
    )(router_logits_t)


def _dispatch_body(pos_hbm, wts_hbm, tok_hbm, scale_hbm,
                   pos_v, wts_v, tok_v, scl_v, semi, semw):
    wid = lax.axis_index("c") * NS + lax.axis_index("s")
    lo = wid * GPT
    cpp = pltpu.async_copy(pos_hbm, pos_v, semi)
    cpw = pltpu.async_copy(wts_hbm, wts_v, semi)

    zi = jnp.zeros((16,), jnp.int32)
    zf = jnp.zeros((16,), jnp.float32)
    for q in range(GPT // 16):
        tok_v[pl.ds(q * 16, 16)] = zi
        scl_v[pl.ds(q * 16, 16)] = zf
    cpp.wait()
    cpw.wait()

    lane = lax.broadcasted_iota(jnp.int32, (16,), 0)

    @pl.loop(0, TK // 16, unroll=16)
    def _scatter(q):
        sr = pos_v[pl.ds(q * 16, 16)]
        tv = (q * 16 + lane) & (T - 1)
        m = (sr >= lo) & (sr < lo + GPT)
        li = sr - lo
        plsc.store_scatter(tok_v, [li], tv, mask=m)
        plsc.store_scatter(scl_v, [li], wts_v[pl.ds(q * 16, 16)], mask=m)

    w0 = pltpu.async_copy(tok_v, tok_hbm.at[pl.ds(lo, GPT)], semw)
    w1 = pltpu.async_copy(scl_v, scale_hbm.at[pl.ds(lo, GPT)], semw)
    w0.wait()
    w1.wait()


def _dispatch(pos_flat, wts_flat):
    mesh = plsc.VectorSubcoreMesh(core_axis_name="c", subcore_axis_name="s")
    f = pl.kernel(
        _dispatch_body,
        out_type=(
            jax.ShapeDtypeStruct((RP,), jnp.int32),
            jax.ShapeDtypeStruct((RP,), jnp.float32),
        ),
        mesh=mesh,
        scratch_types=(
            pltpu.VMEM((TK,), jnp.int32),
            pltpu.VMEM((TK,), jnp.float32),
            pltpu.VMEM((GPT,), jnp.int32),
            pltpu.VMEM((GPT,), jnp.float32),
            pltpu.SemaphoreType.DMA,
            pltpu.SemaphoreType.DMA,
        ),
        compiler_params=pltpu.CompilerParams(needs_layout_passes=False),
    )
    return f(pos_flat, wts_flat)


def _mm_body(te_ref, tok_ref, scale_ref, x_ref, w13_ref, w2_ref, b13_ref,
             b2_ref, out_ref, w13b_ref, w2b_ref):
    i = pl.program_id(0)
    prev = te_ref[jnp.maximum(i - 1, 0)]

    @pl.when((i == 0) | (te_ref[i] != prev))
    def _cast_weights():
        w13b_ref[...] = w13_ref[0].astype(jnp.bfloat16)
        w2b_ref[...] = w2_ref[0].astype(jnp.bfloat16)

    sel = (tok_ref[...] == lax.broadcasted_iota(jnp.int32, (TM, T), 1))
    xs = lax.dot_general(sel.astype(jnp.bfloat16), x_ref[...],
                         (((1,), (0,)), ((), ())),
                         preferred_element_type=jnp.float32)
    h = lax.dot_general(xs.astype(jnp.bfloat16), w13b_ref[...],
                        (((1,), (1,)), ((), ())),
                        preferred_element_type=jnp.float32)
    h = h + b13_ref[0]
    gate = h[:, :II]
    up = h[:, II:]
    act = gate * jax.nn.sigmoid(ALPHA * gate) * (up + BETA)
    y = lax.dot_general(act.astype(jnp.bfloat16), w2b_ref[...],
                        (((1,), (1,)), ((), ())),
                        preferred_element_type=jnp.float32)
    out_ref[...] = (y + b2_ref[0]) * scale_ref[...]


def _mm(te, tok, scale, x_bf, w13, w2, w13_bias, w2_bias):
    grid_spec = pltpu.PrefetchScalarGridSpec(
        num_scalar_prefetch=1,
        grid=(NT,),
        in_specs=[
            pl.BlockSpec((TM, 1), lambda i, te_r: (i, 0)),
            pl.BlockSpec((TM, 1), lambda i, te_r: (i, 0)),
            pl.BlockSpec((T, H), lambda i, te_r: (0, 0)),
            pl.BlockSpec((1, 2 * II, H), lambda i, te_r: (te_r[i], 0, 0)),
            pl.BlockSpec((1, H, II), lambda i, te_r: (te_r[i], 0, 0)),
            pl.BlockSpec((1, 1, 2 * II), lambda i, te_r: (te_r[i], 0, 0)),
            pl.BlockSpec((1, 1, H), lambda i, te_r: (te_r[i], 0, 0)),
        ],
        out_specs=pl.BlockSpec((TM, H), lambda i, te_r: (i, 0)),
        scratch_shapes=[
            pltpu.VMEM((2 * II, H), jnp.bfloat16),
            pltpu.VMEM((H, II), jnp.bfloat16),
        ],
    )
    return pl.pallas_call(
        _mm_body,
        grid_spec=grid_spec,
        out_shape=jax.ShapeDtypeStruct((RP, H), jnp.float32),
    )(te, tok, scale, x_bf, w13, w2, w13_bias.reshape(E, 1, 2 * II),
      w2_bias.reshape(E, 1, H))


def _combine_body(pos_hbm, ys_hbm, out_hbm, pv0, pv1, i0a, i0b, i1a, i1b,
                  b0a, b0b, b1a, b1b, semg, semw):
    wid = lax.axis_index("c") * NS + lax.axis_index("s")
    tok0 = wid * TPT
    pltpu.sync_copy(pos_hbm.at[pl.ds(tok0, TPT)], pv0)
    pltpu.sync_copy(pos_hbm.at[pl.ds(T + tok0, TPT)], pv1)
    NCH = TPT // 16
    b0 = [b0a, b0b]
    b1 = [b1a, b1b]
    i0 = [i0a, i0b]
    i1 = [i1a, i1b]
    g0 = [None] * NCH
    g1 = [None] * NCH
    w = [None] * NCH

    def issue(ch):
        p = ch % 2
        i0[p][...] = pv0[pl.ds(ch * 16, 16)]
        i1[p][...] = pv1[pl.ds(ch * 16, 16)]
        g0[ch] = pltpu.async_copy(ys_hbm.at[i0[p]], b0[p], semg)
        g1[ch] = pltpu.async_copy(ys_hbm.at[i1[p]], b1[p], semg)

    issue(0)
    issue(1)
    for ch in range(NCH):
        p = ch % 2
        if ch >= 2:
            w[ch - 2].wait()
            issue(ch)
        g0[ch].wait()
        g1[ch].wait()
        bb0 = b0[p]
        bb1 = b1[p]

        @pl.loop(0, 16 * (H // 16), unroll=8)
        def _add(q):
            r = lax.shift_right_logical(q, 6)
            c = lax.rem(q, H // 16)
            bb0[r, pl.ds(c * 16, 16)] = (bb0[r, pl.ds(c * 16, 16)]
                                         + bb1[r, pl.ds(c * 16, 16)])

        w[ch] = pltpu.async_copy(bb0, out_hbm.at[pl.ds(tok0 + ch * 16, 16)],
                                 semw)
    w[NCH - 2].wait()
    w[NCH - 1].wait()


def _combine(pos_flat, ys):
    mesh = plsc.VectorSubcoreMesh(core_axis_name="c", subcore_axis_name="s")
    f = pl.kernel(
        _combine_body,
        out_type=jax.ShapeDtypeStruct((T, H), jnp.float32),
        mesh=mesh,
        scratch_types=(
            pltpu.VMEM((TPT,), jnp.int32),
            pltpu.VMEM((TPT,), jnp.int32),
            pltpu.VMEM((16,), jnp.int32),
            pltpu.VMEM((16,), jnp.int32),
            pltpu.VMEM((16,), jnp.int32),
            pltpu.VMEM((16,), jnp.int32),
            pltpu.VMEM((16, H), jnp.float32),
            pltpu.VMEM((16, H), jnp.float32),
            pltpu.VMEM((16, H), jnp.float32),
            pltpu.VMEM((16, H), jnp.float32),
            pltpu.SemaphoreType.DMA,
            pltpu.SemaphoreType.DMA,
        ),
        compiler_params=pltpu.CompilerParams(needs_layout_passes=False),
    )
    return f(pos_flat, ys)


def kernel(x, router_logits, w13, w2, w13_bias, w2_bias):
    pos, wts, te = _routing(router_logits.T)
    pos_flat = pos.reshape(TK)
    wts_flat = wts.reshape(TK)
    tok, scale = _dispatch(pos_flat, wts_flat)
    ys = _mm(te.reshape(NT), tok.reshape(RP, 1), scale.reshape(RP, 1),
             x.astype(jnp.bfloat16), w13, w2, w13_bias, w2_bias)
    return _combine(pos_flat, ys)
